# baseline (speedup 1.0000x reference)
import functools

import jax
import jax.numpy as jnp
from jax import lax
from jax.experimental import pallas as pl
from jax.experimental.pallas import tpu as pltpu
from jax.experimental.pallas import tpu_sc as plsc

NC = 2
NS = 16
LANES = 16
CH = 64


def _silu(x):
    return x * (1.0 / (1.0 + jnp.exp(-x)))


def _make_sc_edge(Np, Ep, D, with_deg, interpret=False):
    NW = NC * NS
    per_w = Ep // NW
    n_chunks = per_w // CH
    rpt = Np // NS
    nzc = rpt // CH
    mesh = plsc.VectorSubcoreMesh(core_axis_name="c", subcore_axis_name="s")

    @functools.partial(
        pl.kernel,
        out_type=(
            jax.ShapeDtypeStruct((NC * Np, D), jnp.float32),
            jax.ShapeDtypeStruct((NW, Np), jnp.float32),
        ),
        mesh=mesh,
        scratch_types=[
            pltpu.VMEM((CH,), jnp.int32),
            pltpu.VMEM((CH,), jnp.int32),
            pltpu.VMEM((CH, D), jnp.float32),
            pltpu.VMEM((CH, D), jnp.float32),
            pltpu.VMEM((CH, D), jnp.float32),
            pltpu.VMEM((CH, D), jnp.float32),
            pltpu.VMEM((Np,), jnp.float32),
            pltpu.VMEM_SHARED((Np, D), jnp.float32),
            pltpu.SemaphoreType.DMA,
            pltpu.SemaphoreType.DMA,
        ],
        compiler_params=pltpu.CompilerParams(needs_layout_passes=False),
        interpret=interpret,
    )
    def sc_edge(a_hbm, b_hbm, c_hbm, src_hbm, dst_hbm, out_hbm, deg_hbm,
                si, di, ab, bb, cb, mb, dv, acc, sem_i, sem_g):
        c = lax.axis_index("c")
        s = lax.axis_index("s")
        wid = s * NC + c
        base = wid * per_w

        zero16 = jnp.zeros((LANES,), jnp.float32)

        def zrow(r, carry):
            for k in range(D // LANES):
                mb[r, pl.ds(k * LANES, LANES)] = zero16
            return carry
        lax.fori_loop(0, CH, zrow, None)
        for j in range(nzc):
            pltpu.sync_copy(mb, acc.at[pl.ds(s * rpt + j * CH, CH)])

        if with_deg:
            def zdeg(i, carry):
                dv[pl.ds(i * LANES, LANES)] = zero16
                return carry
            lax.fori_loop(0, Np // LANES, zdeg, None)

        plsc.subcore_barrier()

        def chunk(j, carry):
            off = base + j * CH
            cpi = pltpu.async_copy(src_hbm.at[pl.ds(off, CH)], si, sem_i)
            cpd = pltpu.async_copy(dst_hbm.at[pl.ds(off, CH)], di, sem_i)
            cpc = pltpu.async_copy(c_hbm.at[pl.ds(off, CH)], cb, sem_i)
            cpi.wait()
            cpd.wait()
            cpa = pltpu.async_copy(a_hbm.at[si], ab, sem_g)
            cpb = pltpu.async_copy(b_hbm.at[di], bb, sem_g)

            if with_deg:
                for k in range(CH // LANES):
                    dvec = di[pl.ds(k * LANES, LANES)]
                    cnt, lastm = plsc.scan_count(dvec)
                    plsc.addupdate_scatter(
                        dv, [dvec], cnt.astype(jnp.float32), mask=lastm)

            cpc.wait()
            cpa.wait()
            cpb.wait()

            def row(r, inner):
                for k in range(D // LANES):
                    sl = pl.ds(k * LANES, LANES)
                    t = ab[r, sl] + bb[r, sl] + cb[r, sl]
                    mb[r, sl] = t * (1.0 / (1.0 + jnp.exp(-t)))
                return inner
            lax.fori_loop(0, CH, row, None)

            pltpu.sync_copy(mb, acc.at[di], add=True)
            return carry
        lax.fori_loop(0, n_chunks, chunk, None)

        if with_deg:
            pltpu.sync_copy(dv, deg_hbm.at[wid])
        plsc.subcore_barrier()
        for j in range(nzc):
            sl = pl.ds(s * rpt + j * CH, CH)
            pltpu.sync_copy(acc.at[sl], out_hbm.at[pl.ds(c * Np + s * rpt + j * CH, CH)])

    return sc_edge


def _edge_bias_const(ea_p, w_attr, b1e2, Ep, ED, D):
    BLK = NC * NS * CH

    def body(ea_ref, w_ref, b_ref, o_ref):
        o_ref[...] = (
            jnp.dot(ea_ref[...], w_ref[...], preferred_element_type=jnp.float32)
            + b_ref[...]
        )

    return pl.pallas_call(
        body,
        grid=(Ep // BLK,),
        in_specs=[
            pl.BlockSpec((BLK, ED), lambda i: (i, 0)),
            pl.BlockSpec((ED, D), lambda i: (0, 0)),
            pl.BlockSpec((1, D), lambda i: (0, 0)),
        ],
        out_specs=pl.BlockSpec((BLK, D), lambda i: (i, 0)),
        out_shape=jax.ShapeDtypeStruct((Ep, D), jnp.float32),
    )(ea_p, w_attr, b1e2)


def _ab_tables(h_p, w_src, w_dst, Np, D):
    RB = 1280

    def body(h_ref, ws_ref, wd_ref, a_ref, b_ref):
        hh = h_ref[...]
        a_ref[...] = jnp.dot(hh, ws_ref[...], preferred_element_type=jnp.float32)
        b_ref[...] = jnp.dot(hh, wd_ref[...], preferred_element_type=jnp.float32)

    return pl.pallas_call(
        body,
        grid=(Np // RB,),
        in_specs=[
            pl.BlockSpec((RB, D), lambda i: (i, 0)),
            pl.BlockSpec((D, D), lambda i: (0, 0)),
            pl.BlockSpec((D, D), lambda i: (0, 0)),
        ],
        out_specs=[
            pl.BlockSpec((RB, D), lambda i: (i, 0)),
            pl.BlockSpec((RB, D), lambda i: (i, 0)),
        ],
        out_shape=[
            jax.ShapeDtypeStruct((Np, D), jnp.float32),
            jax.ShapeDtypeStruct((Np, D), jnp.float32),
        ],
    )(h_p, w_src, w_dst)


def _node_update(S2, deg_w, h_p, W2e, b2e2, W1n, b1n2, W2n, b2n2, w_src, w_dst,
                 Np, D, NW, compute_ab):
    RB = 1280

    def body(s_ref, deg_ref, h_ref, w2e_ref, b2e_ref, w1n_ref, b1n_ref,
             w2n_ref, b2n_ref, ws_ref, wd_ref, hn_ref, *ab_refs):
        S = s_ref[0] + s_ref[1]
        deg = jnp.sum(deg_ref[...], axis=0)[:, None]
        degc = jnp.maximum(deg, 1.0)
        mask = (deg > 0.0).astype(jnp.float32)
        agg = (
            jnp.dot(S, w2e_ref[...], preferred_element_type=jnp.float32) / degc
            + b2e_ref[...] * mask
        )
        hh = h_ref[...]
        z = (
            jnp.dot(hh, w1n_ref[:D, :], preferred_element_type=jnp.float32)
            + jnp.dot(agg, w1n_ref[D:, :], preferred_element_type=jnp.float32)
            + b1n_ref[...]
        )
        hn = hh + jnp.dot(_silu(z), w2n_ref[...],
                          preferred_element_type=jnp.float32) + b2n_ref[...]
        hn_ref[...] = hn
        if compute_ab:
            a_ref, b_ref = ab_refs
            a_ref[...] = jnp.dot(hn, ws_ref[...], preferred_element_type=jnp.float32)
            b_ref[...] = jnp.dot(hn, wd_ref[...], preferred_element_type=jnp.float32)

    n_out = 3 if compute_ab else 1
    return pl.pallas_call(
        body,
        grid=(Np // RB,),
        in_specs=[
            pl.BlockSpec((2, RB, D), lambda i: (0, i, 0)),
            pl.BlockSpec((NW, RB), lambda i: (0, i)),
            pl.BlockSpec((RB, D), lambda i: (i, 0)),
            pl.BlockSpec((D, D), lambda i: (0, 0)),
            pl.BlockSpec((1, D), lambda i: (0, 0)),
            pl.BlockSpec((2 * D, D), lambda i: (0, 0)),
            pl.BlockSpec((1, D), lambda i: (0, 0)),
            pl.BlockSpec((D, D), lambda i: (0, 0)),
            pl.BlockSpec((1, D), lambda i: (0, 0)),
            pl.BlockSpec((D, D), lambda i: (0, 0)),
            pl.BlockSpec((D, D), lambda i: (0, 0)),
        ],
        out_specs=[pl.BlockSpec((RB, D), lambda i: (i, 0))] * n_out,
        out_shape=[jax.ShapeDtypeStruct((Np, D), jnp.float32)] * n_out,
    )(S2, deg_w, h_p, W2e, b2e2, W1n, b1n2, W2n, b2n2, w_src, w_dst)


def kernel(node_state, edge_index, edge_attr, W1e, b1e, W2e, b2e,
           W1n, b1n, W2n, b2n):
    N, D = node_state.shape
    E = edge_index.shape[1]
    ED = edge_attr.shape[1]
    NW = NC * NS

    Np = -(-(N + 1) // (NS * CH)) * (NS * CH)
    Ep = -(-E // (NW * CH)) * (NW * CH)

    src = edge_index[0].astype(jnp.int32)
    dst = edge_index[1].astype(jnp.int32)
    src_p = jnp.concatenate([src, jnp.zeros((Ep - E,), jnp.int32)])
    dst_p = jnp.concatenate([dst, jnp.full((Ep - E,), N, jnp.int32)])
    ea_p = jnp.pad(edge_attr, ((0, Ep - E), (0, 0)))
    h_p = jnp.pad(node_state, ((0, Np - N), (0, 0)))

    w_src = W1e[:D]
    w_dst = W1e[D:2 * D]
    w_attr = W1e[2 * D:]
    b1e2 = b1e.reshape(1, D)
    b2e2 = b2e.reshape(1, D)
    b1n2 = b1n.reshape(1, D)
    b2n2 = b2n.reshape(1, D)

    C = _edge_bias_const(ea_p, w_attr, b1e2, Ep, ED, D)
    sc_edge_deg = _make_sc_edge(Np, Ep, D, with_deg=True)
    sc_edge = _make_sc_edge(Np, Ep, D, with_deg=False)

    A, B = _ab_tables(h_p, w_src, w_dst, Np, D)

    S, deg_w = sc_edge_deg(A, B, C, src_p, dst_p)
    S = S.reshape(NC, Np, D)
    h_p, A, B = _node_update(S, deg_w, h_p, W2e, b2e2, W1n, b1n2, W2n, b2n2,
                             w_src, w_dst, Np, D, NW, compute_ab=True)

    S, _ = sc_edge(A, B, C, src_p, dst_p)
    S = S.reshape(NC, Np, D)
    (h_p,) = _node_update(S, deg_w, h_p, W2e, b2e2, W1n, b1n2, W2n, b2n2,
                          w_src, w_dst, Np, D, NW, compute_ab=False)

    return h_p[:N]


# speedup vs baseline: 1.0487x; 1.0487x over previous
import functools

import jax
import jax.numpy as jnp
from jax import lax
from jax.experimental import pallas as pl
from jax.experimental.pallas import tpu as pltpu
from jax.experimental.pallas import tpu_sc as plsc

NC = 2
NS = 16
LANES = 16
CH = 64


def _silu(x):
    return x * (1.0 / (1.0 + jnp.exp(-x)))


def _make_sc_edge(Np, Ep, D, with_deg, interpret=False):
    NW = NC * NS
    per_w = Ep // NW
    n_chunks = per_w // CH
    rpt = Np // NS
    nzc = rpt // CH
    mesh = plsc.VectorSubcoreMesh(core_axis_name="c", subcore_axis_name="s")

    @functools.partial(
        pl.kernel,
        out_type=(
            jax.ShapeDtypeStruct((NC * Np, D), jnp.float32),
            jax.ShapeDtypeStruct((NW, Np), jnp.float32),
        ),
        mesh=mesh,
        scratch_types=[
            pltpu.VMEM((CH,), jnp.int32),
            pltpu.VMEM((CH,), jnp.int32),
            pltpu.VMEM((CH, D), jnp.float32),
            pltpu.VMEM((CH, D), jnp.float32),
            pltpu.VMEM((CH, D), jnp.float32),
            pltpu.VMEM((CH, D), jnp.float32),
            pltpu.VMEM((Np,), jnp.float32),
            pltpu.VMEM_SHARED((Np, D), jnp.float32),
            pltpu.SemaphoreType.DMA,
            pltpu.SemaphoreType.DMA,
        ],
        compiler_params=pltpu.CompilerParams(needs_layout_passes=False),
        interpret=interpret,
    )
    def sc_edge(a_hbm, b_hbm, c_hbm, src_hbm, dst_hbm, out_hbm, deg_hbm,
                si, di, ab, bb, cb, mb, dv, acc, sem_i, sem_g):
        c = lax.axis_index("c")
        s = lax.axis_index("s")
        wid = s * NC + c
        base = wid * per_w

        zero16 = jnp.zeros((LANES,), jnp.float32)

        def zrow(r, carry):
            for k in range(D // LANES):
                mb[r, pl.ds(k * LANES, LANES)] = zero16
            return carry
        lax.fori_loop(0, CH, zrow, None)
        for j in range(nzc):
            pltpu.sync_copy(mb, acc.at[pl.ds(s * rpt + j * CH, CH)])

        if with_deg:
            def zdeg(i, carry):
                dv[pl.ds(i * LANES, LANES)] = zero16
                return carry
            lax.fori_loop(0, Np // LANES, zdeg, None)

        plsc.subcore_barrier()

        def chunk(j, carry):
            off = base + j * CH
            cpi = pltpu.async_copy(src_hbm.at[pl.ds(off, CH)], si, sem_i)
            cpd = pltpu.async_copy(dst_hbm.at[pl.ds(off, CH)], di, sem_i)
            cpc = pltpu.async_copy(c_hbm.at[pl.ds(off, CH)], cb, sem_i)
            cpi.wait()
            cpd.wait()
            cpa = pltpu.async_copy(a_hbm.at[si], ab, sem_g)
            cpb = pltpu.async_copy(b_hbm.at[di], bb, sem_g)

            if with_deg:
                for k in range(CH // LANES):
                    dvec = di[pl.ds(k * LANES, LANES)]
                    cnt, lastm = plsc.scan_count(dvec)
                    plsc.addupdate_scatter(
                        dv, [dvec], cnt.astype(jnp.float32), mask=lastm)

            cpc.wait()
            cpa.wait()
            cpb.wait()

            def row(r, inner):
                for k in range(D // LANES):
                    sl = pl.ds(k * LANES, LANES)
                    t = ab[r, sl] + bb[r, sl] + cb[r, sl]
                    mb[r, sl] = t * (1.0 / (1.0 + jnp.exp(-t)))
                return inner
            lax.fori_loop(0, CH, row, None)

            pltpu.sync_copy(mb, acc.at[di], add=True)
            return carry
        lax.fori_loop(0, n_chunks, chunk, None)

        if with_deg:
            pltpu.sync_copy(dv, deg_hbm.at[wid])
        plsc.subcore_barrier()
        for j in range(nzc):
            sl = pl.ds(s * rpt + j * CH, CH)
            pltpu.sync_copy(acc.at[sl], out_hbm.at[pl.ds(c * Np + s * rpt + j * CH, CH)])

    return sc_edge


def _edge_bias_const(ea_p, w_attr, b1e2, Ep, ED, D):
    BLK = NC * NS * CH
    for d in range(16384, 7, -8):
        if Ep % d == 0:
            BLK = d
            break

    def body(ea_ref, w_ref, b_ref, o_ref):
        o_ref[...] = (
            jnp.dot(ea_ref[...], w_ref[...], preferred_element_type=jnp.float32)
            + b_ref[...]
        )

    return pl.pallas_call(
        body,
        grid=(Ep // BLK,),
        in_specs=[
            pl.BlockSpec((BLK, ED), lambda i: (i, 0)),
            pl.BlockSpec((ED, D), lambda i: (0, 0)),
            pl.BlockSpec((1, D), lambda i: (0, 0)),
        ],
        out_specs=pl.BlockSpec((BLK, D), lambda i: (i, 0)),
        out_shape=jax.ShapeDtypeStruct((Ep, D), jnp.float32),
    )(ea_p, w_attr, b1e2)


def _ab_tables(h_p, w_src, w_dst, Np, D):
    RB = 1280

    def body(h_ref, ws_ref, wd_ref, a_ref, b_ref):
        hh = h_ref[...]
        a_ref[...] = jnp.dot(hh, ws_ref[...], preferred_element_type=jnp.float32)
        b_ref[...] = jnp.dot(hh, wd_ref[...], preferred_element_type=jnp.float32)

    return pl.pallas_call(
        body,
        grid=(Np // RB,),
        in_specs=[
            pl.BlockSpec((RB, D), lambda i: (i, 0)),
            pl.BlockSpec((D, D), lambda i: (0, 0)),
            pl.BlockSpec((D, D), lambda i: (0, 0)),
        ],
        out_specs=[
            pl.BlockSpec((RB, D), lambda i: (i, 0)),
            pl.BlockSpec((RB, D), lambda i: (i, 0)),
        ],
        out_shape=[
            jax.ShapeDtypeStruct((Np, D), jnp.float32),
            jax.ShapeDtypeStruct((Np, D), jnp.float32),
        ],
    )(h_p, w_src, w_dst)


def _node_update(S2, deg_w, h_p, W2e, b2e2, W1n, b1n2, W2n, b2n2, w_src, w_dst,
                 Np, D, NW, compute_ab):
    RB = 1280

    def body(s_ref, deg_ref, h_ref, w2e_ref, b2e_ref, w1n_ref, b1n_ref,
             w2n_ref, b2n_ref, ws_ref, wd_ref, hn_ref, *ab_refs):
        S = s_ref[0] + s_ref[1]
        deg = jnp.sum(deg_ref[...], axis=0)[:, None]
        degc = jnp.maximum(deg, 1.0)
        mask = (deg > 0.0).astype(jnp.float32)
        agg = (
            jnp.dot(S, w2e_ref[...], preferred_element_type=jnp.float32) / degc
            + b2e_ref[...] * mask
        )
        hh = h_ref[...]
        z = (
            jnp.dot(hh, w1n_ref[:D, :], preferred_element_type=jnp.float32)
            + jnp.dot(agg, w1n_ref[D:, :], preferred_element_type=jnp.float32)
            + b1n_ref[...]
        )
        hn = hh + jnp.dot(_silu(z), w2n_ref[...],
                          preferred_element_type=jnp.float32) + b2n_ref[...]
        hn_ref[...] = hn
        if compute_ab:
            a_ref, b_ref = ab_refs
            a_ref[...] = jnp.dot(hn, ws_ref[...], preferred_element_type=jnp.float32)
            b_ref[...] = jnp.dot(hn, wd_ref[...], preferred_element_type=jnp.float32)

    n_out = 3 if compute_ab else 1
    return pl.pallas_call(
        body,
        grid=(Np // RB,),
        in_specs=[
            pl.BlockSpec((2, RB, D), lambda i: (0, i, 0)),
            pl.BlockSpec((NW, RB), lambda i: (0, i)),
            pl.BlockSpec((RB, D), lambda i: (i, 0)),
            pl.BlockSpec((D, D), lambda i: (0, 0)),
            pl.BlockSpec((1, D), lambda i: (0, 0)),
            pl.BlockSpec((2 * D, D), lambda i: (0, 0)),
            pl.BlockSpec((1, D), lambda i: (0, 0)),
            pl.BlockSpec((D, D), lambda i: (0, 0)),
            pl.BlockSpec((1, D), lambda i: (0, 0)),
            pl.BlockSpec((D, D), lambda i: (0, 0)),
            pl.BlockSpec((D, D), lambda i: (0, 0)),
        ],
        out_specs=[pl.BlockSpec((RB, D), lambda i: (i, 0))] * n_out,
        out_shape=[jax.ShapeDtypeStruct((Np, D), jnp.float32)] * n_out,
    )(S2, deg_w, h_p, W2e, b2e2, W1n, b1n2, W2n, b2n2, w_src, w_dst)


def kernel(node_state, edge_index, edge_attr, W1e, b1e, W2e, b2e,
           W1n, b1n, W2n, b2n):
    N, D = node_state.shape
    E = edge_index.shape[1]
    ED = edge_attr.shape[1]
    NW = NC * NS

    Np = -(-(N + 1) // (NS * CH)) * (NS * CH)
    Ep = -(-E // (NW * CH)) * (NW * CH)

    src = edge_index[0].astype(jnp.int32)
    dst = edge_index[1].astype(jnp.int32)
    src_p = jnp.concatenate([src, jnp.zeros((Ep - E,), jnp.int32)])
    dst_p = jnp.concatenate([dst, jnp.full((Ep - E,), N, jnp.int32)])
    ea_p = jnp.pad(edge_attr, ((0, Ep - E), (0, 0)))
    h_p = jnp.pad(node_state, ((0, Np - N), (0, 0)))

    w_src = W1e[:D]
    w_dst = W1e[D:2 * D]
    w_attr = W1e[2 * D:]
    b1e2 = b1e.reshape(1, D)
    b2e2 = b2e.reshape(1, D)
    b1n2 = b1n.reshape(1, D)
    b2n2 = b2n.reshape(1, D)

    C = _edge_bias_const(ea_p, w_attr, b1e2, Ep, ED, D)
    sc_edge_deg = _make_sc_edge(Np, Ep, D, with_deg=True)
    sc_edge = _make_sc_edge(Np, Ep, D, with_deg=False)

    A, B = _ab_tables(h_p, w_src, w_dst, Np, D)

    S, deg_w = sc_edge_deg(A, B, C, src_p, dst_p)
    S = S.reshape(NC, Np, D)
    h_p, A, B = _node_update(S, deg_w, h_p, W2e, b2e2, W1n, b1n2, W2n, b2n2,
                             w_src, w_dst, Np, D, NW, compute_ab=True)

    S, _ = sc_edge(A, B, C, src_p, dst_p)
    S = S.reshape(NC, Np, D)
    (h_p,) = _node_update(S, deg_w, h_p, W2e, b2e2, W1n, b1n2, W2n, b2n2,
                          w_src, w_dst, Np, D, NW, compute_ab=False)

    return h_p[:N]
